# baseline (speedup 1.0000x reference)
import jax
import jax.numpy as jnp
from jax.experimental import pallas as pl

B, N, DIMS = 4, 4096, 2
INPUT_UNITS = 64
UNITS = 128
K = 16
NEIGH = K + 1
LRELU_ALPHA = 0.2


def _knn_idx(pc, k):
    d2 = jnp.sum((pc[:, :, None, :] - pc[:, None, :, :]) ** 2, axis=-1)
    _, idx = jax.lax.top_k(-d2, k)
    return idx


def _gather_neighbours(x, n_idx):
    b, n, k = n_idx.shape
    def g(xx, ii):
        return xx[ii.reshape(-1)].reshape(n, k, xx.shape[-1])
    return jax.vmap(g)(x, n_idx)


def _m_lin_fit(pts):
    x = pts[..., 0]
    y = pts[..., 1]
    k = pts.shape[-2]
    sx = jnp.sum(x, -1)
    sy = jnp.sum(y, -1)
    num = k * jnp.sum(x * y, -1) - sx * sy
    den = k * jnp.sum(x * x, -1) - sx ** 2
    return (num / (den + 1e-8))[..., None, None]


def _pearson(pts):
    x = pts[..., 0]
    y = pts[..., 1]
    xm = x - x.mean(-1, keepdims=True)
    ym = y - y.mean(-1, keepdims=True)
    num = jnp.sum(xm * ym, -1)
    den = jnp.sqrt(jnp.sum(xm * xm, -1) * jnp.sum(ym * ym, -1)) + 1e-8
    return (num / den)[..., None, None]


def _locse(pc, feats, n_idx, W, b):
    n_points = _gather_neighbours(pc, n_idx)
    norms = jnp.sqrt(jnp.sum(n_points ** 2, axis=-1, keepdims=True) + 1e-12)
    rpbns = []
    for i in range(NEIGH):
        current = n_points[:, :, i:i + 1]
        diff = current - n_points
        rpbns.append(jnp.concatenate([diff, norms], axis=-1))
    rppe = jnp.concatenate([n_points] + rpbns, axis=-1)
    ggf = jnp.concatenate([_m_lin_fit(n_points), _pearson(n_points)], axis=-1)
    r = jax.nn.relu(jnp.einsum('bnkc,cd->bnkd', rppe, W) + b)
    n_feats = _gather_neighbours(feats, n_idx)
    return jnp.concatenate([n_feats, r], axis=-1), ggf


def _att_pool(n_feats, ggf, Ws, bs, Wf, bf):
    scores = jax.nn.softmax(jnp.einsum('bnkc,cd->bnkd', n_feats, Ws) + bs, axis=-1)
    attention = jnp.sum(n_feats * scores, axis=-2, keepdims=True)
    cat = jnp.concatenate([attention, ggf], axis=-1)
    out = jax.nn.relu(jnp.einsum('bnkc,cd->bnkd', cat, Wf) + bf)
    return out.reshape(out.shape[0], out.shape[1], -1)


def _final_kernel(y_ref, feats_ref, w1_ref, b1_ref, wsk_ref, bsk_ref, o_ref):
    y = jnp.maximum(y_ref[0] @ w1_ref[...] + b1_ref[...], 0.0)
    skip = jnp.maximum(feats_ref[0] @ wsk_ref[...] + bsk_ref[...], 0.0)
    z = y + skip
    o_ref[0] = jnp.where(z >= 0, z, LRELU_ALPHA * z)


def kernel(pc, feats, W0, b0, Wl0, bl0, Ws0, bs0, Wf0, bf0, Wl1, bl1, Ws1, bs1, Wf1, bf1, W1, b1, Wskip, bskip):
    n_idx = _knn_idx(pc, NEIGH)
    y = jax.nn.relu(feats @ W0 + b0)
    nf0, ggf0 = _locse(pc, y, n_idx, Wl0, bl0)
    y = _att_pool(nf0, ggf0, Ws0, bs0, Wf0, bf0)
    nf1, ggf1 = _locse(pc, y, n_idx, Wl1, bl1)
    y = _att_pool(nf1, ggf1, Ws1, bs1, Wf1, bf1)

    TN = 512
    out = pl.pallas_call(
        _final_kernel,
        grid=(B, N // TN),
        in_specs=[
            pl.BlockSpec((1, TN, UNITS), lambda b_, n_: (b_, n_, 0)),
            pl.BlockSpec((1, TN, INPUT_UNITS), lambda b_, n_: (b_, n_, 0)),
            pl.BlockSpec((UNITS, UNITS), lambda b_, n_: (0, 0)),
            pl.BlockSpec((UNITS,), lambda b_, n_: (0,)),
            pl.BlockSpec((INPUT_UNITS, UNITS), lambda b_, n_: (0, 0)),
            pl.BlockSpec((UNITS,), lambda b_, n_: (0,)),
        ],
        out_specs=pl.BlockSpec((1, TN, UNITS), lambda b_, n_: (b_, n_, 0)),
        out_shape=jax.ShapeDtypeStruct((B, N, UNITS), jnp.float32),
    )(y, feats, W1, b1, Wskip, bskip)
    return out


# speedup vs baseline: 1.7593x; 1.7593x over previous
import jax
import jax.numpy as jnp
from jax.experimental import pallas as pl

B, N, DIMS = 4, 4096, 2
INPUT_UNITS = 64
UNITS = 128
K = 16
NEIGH = K + 1
LRELU_ALPHA = 0.2


TN_KNN = 512


def _knn_mlp0_kernel(pxc_ref, pyc_ref, pxr_ref, pyr_ref, feats_ref,
                     w0_ref, b0_ref, wsk_ref, bsk_ref,
                     idx_ref, y0_ref, skip_ref):
    xc = pxc_ref[0]
    yc = pyc_ref[0]
    xr = pxr_ref[0]
    yr = pyr_ref[0]
    dx = xc - xr
    dy = yc - yr
    d2 = dx * dx + dy * dy
    iota = jax.lax.broadcasted_iota(jnp.int32, (TN_KNN, N), 1)
    cols = []
    for _ in range(NEIGH):
        m = jnp.min(d2, axis=1, keepdims=True)
        eq = d2 == m
        iv = jnp.where(eq, iota, N)
        idx = jnp.min(iv, axis=1, keepdims=True)
        d2 = jnp.where(iota == idx, 3.4e38, d2)
        cols.append(idx)
    idx_ref[0] = jnp.concatenate(cols, axis=1)
    f = feats_ref[0]
    y0_ref[0] = jnp.maximum(f @ w0_ref[...] + b0_ref[...], 0.0)
    skip_ref[0] = jnp.maximum(f @ wsk_ref[...] + bsk_ref[...], 0.0)


def _knn_mlp0(pc, feats, W0, b0, Wskip, bskip):
    pxc = pc[:, :, 0:1]
    pyc = pc[:, :, 1:2]
    pxr = pc[:, :, 0].reshape(B, 1, N)
    pyr = pc[:, :, 1].reshape(B, 1, N)
    G = N // TN_KNN
    return pl.pallas_call(
        _knn_mlp0_kernel,
        grid=(B, G),
        in_specs=[
            pl.BlockSpec((1, TN_KNN, 1), lambda b_, t: (b_, t, 0)),
            pl.BlockSpec((1, TN_KNN, 1), lambda b_, t: (b_, t, 0)),
            pl.BlockSpec((1, 1, N), lambda b_, t: (b_, 0, 0)),
            pl.BlockSpec((1, 1, N), lambda b_, t: (b_, 0, 0)),
            pl.BlockSpec((1, TN_KNN, INPUT_UNITS), lambda b_, t: (b_, t, 0)),
            pl.BlockSpec((INPUT_UNITS, UNITS // 4), lambda b_, t: (0, 0)),
            pl.BlockSpec((UNITS // 4,), lambda b_, t: (0,)),
            pl.BlockSpec((INPUT_UNITS, UNITS), lambda b_, t: (0, 0)),
            pl.BlockSpec((UNITS,), lambda b_, t: (0,)),
        ],
        out_specs=[
            pl.BlockSpec((1, TN_KNN, NEIGH), lambda b_, t: (b_, t, 0)),
            pl.BlockSpec((1, TN_KNN, UNITS // 4), lambda b_, t: (b_, t, 0)),
            pl.BlockSpec((1, TN_KNN, UNITS), lambda b_, t: (b_, t, 0)),
        ],
        out_shape=[
            jax.ShapeDtypeStruct((B, N, NEIGH), jnp.int32),
            jax.ShapeDtypeStruct((B, N, UNITS // 4), jnp.float32),
            jax.ShapeDtypeStruct((B, N, UNITS), jnp.float32),
        ],
    )(pxc, pyc, pxr, pyr, feats, W0, b0, Wskip, bskip)


def _gather_neighbours(x, n_idx):
    b, n, k = n_idx.shape
    def g(xx, ii):
        return xx[ii.reshape(-1)].reshape(n, k, xx.shape[-1])
    return jax.vmap(g)(x, n_idx)


def _m_lin_fit(pts):
    x = pts[..., 0]
    y = pts[..., 1]
    k = pts.shape[-2]
    sx = jnp.sum(x, -1)
    sy = jnp.sum(y, -1)
    num = k * jnp.sum(x * y, -1) - sx * sy
    den = k * jnp.sum(x * x, -1) - sx ** 2
    return (num / (den + 1e-8))[..., None, None]


def _pearson(pts):
    x = pts[..., 0]
    y = pts[..., 1]
    xm = x - x.mean(-1, keepdims=True)
    ym = y - y.mean(-1, keepdims=True)
    num = jnp.sum(xm * ym, -1)
    den = jnp.sqrt(jnp.sum(xm * xm, -1) * jnp.sum(ym * ym, -1)) + 1e-8
    return (num / den)[..., None, None]


def _locse(pc, feats, n_idx, W, b):
    n_points = _gather_neighbours(pc, n_idx)
    norms = jnp.sqrt(jnp.sum(n_points ** 2, axis=-1, keepdims=True) + 1e-12)
    rpbns = []
    for i in range(NEIGH):
        current = n_points[:, :, i:i + 1]
        diff = current - n_points
        rpbns.append(jnp.concatenate([diff, norms], axis=-1))
    rppe = jnp.concatenate([n_points] + rpbns, axis=-1)
    ggf = jnp.concatenate([_m_lin_fit(n_points), _pearson(n_points)], axis=-1)
    r = jax.nn.relu(jnp.einsum('bnkc,cd->bnkd', rppe, W) + b)
    n_feats = _gather_neighbours(feats, n_idx)
    return jnp.concatenate([n_feats, r], axis=-1), ggf


def _att_pool(n_feats, ggf, Ws, bs, Wf, bf):
    scores = jax.nn.softmax(jnp.einsum('bnkc,cd->bnkd', n_feats, Ws) + bs, axis=-1)
    attention = jnp.sum(n_feats * scores, axis=-2, keepdims=True)
    cat = jnp.concatenate([attention, ggf], axis=-1)
    out = jax.nn.relu(jnp.einsum('bnkc,cd->bnkd', cat, Wf) + bf)
    return out.reshape(out.shape[0], out.shape[1], -1)


def _final2_kernel(y_ref, skip_ref, o_ref):
    z = y_ref[0] + skip_ref[0]
    o_ref[0] = jnp.where(z >= 0, z, LRELU_ALPHA * z)


def _final_pallas(y, skip):
    TN = 512
    return pl.pallas_call(
        _final2_kernel,
        grid=(B, N // TN),
        in_specs=[
            pl.BlockSpec((1, TN, UNITS), lambda b_, n_: (b_, n_, 0)),
            pl.BlockSpec((1, TN, UNITS), lambda b_, n_: (b_, n_, 0)),
        ],
        out_specs=pl.BlockSpec((1, TN, UNITS), lambda b_, n_: (b_, n_, 0)),
        out_shape=jax.ShapeDtypeStruct((B, N, UNITS), jnp.float32),
    )(y, skip)


def kernel(pc, feats, W0, b0, Wl0, bl0, Ws0, bs0, Wf0, bf0, Wl1, bl1, Ws1, bs1, Wf1, bf1, W1, b1, Wskip, bskip):
    n_idx, y, skip = _knn_mlp0(pc, feats, W0, b0, Wskip, bskip)
    nf0, ggf0 = _locse(pc, y, n_idx, Wl0, bl0)
    y = _att_pool(nf0, ggf0, Ws0, bs0, Wf0, bf0)
    nf1, ggf1 = _locse(pc, y, n_idx, Wl1, bl1)
    y = _att_pool(nf1, ggf1, Ws1, bs1, Wf1, bf1)
    y = jax.nn.relu(y @ W1 + b1)
    return _final_pallas(y, skip)


# speedup vs baseline: 1.8547x; 1.0542x over previous
import jax
import jax.numpy as jnp
from jax.experimental import pallas as pl

B, N, DIMS = 4, 4096, 2
INPUT_UNITS = 64
UNITS = 128
K = 16
NEIGH = K + 1
LRELU_ALPHA = 0.2


TN_KNN = 512


def _knn_mlp0_kernel(pxc_ref, pyc_ref, pxr_ref, pyr_ref, feats_ref,
                     w0_ref, b0_ref, wsk_ref, bsk_ref,
                     idx_ref, y0_ref, skip_ref):
    xc = pxc_ref[0]
    yc = pyc_ref[0]
    xr = pxr_ref[0]
    yr = pyr_ref[0]
    dx = xc - xr
    dy = yc - yr
    d2 = dx * dx + dy * dy
    iota = jax.lax.broadcasted_iota(jnp.int32, (TN_KNN, N), 1)
    cols = []
    for _ in range(NEIGH):
        m = jnp.min(d2, axis=1, keepdims=True)
        eq = d2 == m
        iv = jnp.where(eq, iota, N)
        idx = jnp.min(iv, axis=1, keepdims=True)
        d2 = jnp.where(iota == idx, 3.4e38, d2)
        cols.append(idx)
    idx_ref[0] = jnp.concatenate(cols, axis=1)
    f = feats_ref[0]
    y0_ref[0] = jnp.maximum(f @ w0_ref[...] + b0_ref[...], 0.0)
    skip_ref[0] = jnp.maximum(f @ wsk_ref[...] + bsk_ref[...], 0.0)


def _knn_mlp0(pc, feats, W0, b0, Wskip, bskip):
    pxc = pc[:, :, 0:1]
    pyc = pc[:, :, 1:2]
    pxr = pc[:, :, 0].reshape(B, 1, N)
    pyr = pc[:, :, 1].reshape(B, 1, N)
    G = N // TN_KNN
    return pl.pallas_call(
        _knn_mlp0_kernel,
        grid=(B, G),
        in_specs=[
            pl.BlockSpec((1, TN_KNN, 1), lambda b_, t: (b_, t, 0)),
            pl.BlockSpec((1, TN_KNN, 1), lambda b_, t: (b_, t, 0)),
            pl.BlockSpec((1, 1, N), lambda b_, t: (b_, 0, 0)),
            pl.BlockSpec((1, 1, N), lambda b_, t: (b_, 0, 0)),
            pl.BlockSpec((1, TN_KNN, INPUT_UNITS), lambda b_, t: (b_, t, 0)),
            pl.BlockSpec((INPUT_UNITS, UNITS // 4), lambda b_, t: (0, 0)),
            pl.BlockSpec((UNITS // 4,), lambda b_, t: (0,)),
            pl.BlockSpec((INPUT_UNITS, UNITS), lambda b_, t: (0, 0)),
            pl.BlockSpec((UNITS,), lambda b_, t: (0,)),
        ],
        out_specs=[
            pl.BlockSpec((1, TN_KNN, NEIGH), lambda b_, t: (b_, t, 0)),
            pl.BlockSpec((1, TN_KNN, UNITS // 4), lambda b_, t: (b_, t, 0)),
            pl.BlockSpec((1, TN_KNN, UNITS), lambda b_, t: (b_, t, 0)),
        ],
        out_shape=[
            jax.ShapeDtypeStruct((B, N, NEIGH), jnp.int32),
            jax.ShapeDtypeStruct((B, N, UNITS // 4), jnp.float32),
            jax.ShapeDtypeStruct((B, N, UNITS), jnp.float32),
        ],
    )(pxc, pyc, pxr, pyr, feats, W0, b0, Wskip, bskip)


def _gather_neighbours(x, n_idx):
    b, n, k = n_idx.shape
    def g(xx, ii):
        return xx[ii.reshape(-1)].reshape(n, k, xx.shape[-1])
    return jax.vmap(g)(x, n_idx)


def _m_lin_fit(pts):
    x = pts[..., 0]
    y = pts[..., 1]
    k = pts.shape[-2]
    sx = jnp.sum(x, -1)
    sy = jnp.sum(y, -1)
    num = k * jnp.sum(x * y, -1) - sx * sy
    den = k * jnp.sum(x * x, -1) - sx ** 2
    return (num / (den + 1e-8))[..., None, None]


def _pearson(pts):
    x = pts[..., 0]
    y = pts[..., 1]
    xm = x - x.mean(-1, keepdims=True)
    ym = y - y.mean(-1, keepdims=True)
    num = jnp.sum(xm * ym, -1)
    den = jnp.sqrt(jnp.sum(xm * xm, -1) * jnp.sum(ym * ym, -1)) + 1e-8
    return (num / den)[..., None, None]


def _locse(pc, feats, n_idx, W, b):
    n_points = _gather_neighbours(pc, n_idx)
    norms = jnp.sqrt(jnp.sum(n_points ** 2, axis=-1, keepdims=True) + 1e-12)
    rpbns = []
    for i in range(NEIGH):
        current = n_points[:, :, i:i + 1]
        diff = current - n_points
        rpbns.append(jnp.concatenate([diff, norms], axis=-1))
    rppe = jnp.concatenate([n_points] + rpbns, axis=-1)
    ggf = jnp.concatenate([_m_lin_fit(n_points), _pearson(n_points)], axis=-1)
    r = jax.nn.relu(jnp.einsum('bnkc,cd->bnkd', rppe, W) + b)
    n_feats = _gather_neighbours(feats, n_idx)
    return jnp.concatenate([n_feats, r], axis=-1), ggf


def _att_pool(n_feats, ggf, Ws, bs, Wf, bf):
    scores = jax.nn.softmax(jnp.einsum('bnkc,cd->bnkd', n_feats, Ws) + bs, axis=-1)
    attention = jnp.sum(n_feats * scores, axis=-2, keepdims=True)
    cat = jnp.concatenate([attention, ggf], axis=-1)
    out = jax.nn.relu(jnp.einsum('bnkc,cd->bnkd', cat, Wf) + bf)
    return out.reshape(out.shape[0], out.shape[1], -1)


TN_L = 512


def _split_locse_w(Wl):
    Ax = Wl[2::3]
    Ay = Wl[3::3]
    Cn = Wl[4::3]
    csum = jnp.sum(Cn, axis=0)
    wdx = Wl[0] - jnp.sum(Ax, axis=0)
    wdy = Wl[1] - jnp.sum(Ay, axis=0)
    return Ax, Ay, csum, wdx, wdy


def _ggf(Xn, Yn):
    k = float(NEIGH)
    sx = jnp.sum(Xn, 1, keepdims=True)
    sy = jnp.sum(Yn, 1, keepdims=True)
    sxy = jnp.sum(Xn * Yn, 1, keepdims=True)
    sxx = jnp.sum(Xn * Xn, 1, keepdims=True)
    mlin = (k * sxy - sx * sy) / (k * sxx - sx * sx + 1e-8)
    xm = Xn - sx / k
    ym = Yn - sy / k
    num = jnp.sum(xm * ym, 1, keepdims=True)
    den = jnp.sqrt(jnp.sum(xm * xm, 1, keepdims=True) *
                   jnp.sum(ym * ym, 1, keepdims=True)) + 1e-8
    return mlin, num / den


def _block_body(Xn, Yn, XT, YT, nyT, Ax, Ay, csum, wdx, wdy, bl, Ws, bs, Wf, bf):
    D = Ax.shape[1]
    base = Xn @ Ax + Yn @ Ay + bl[None, :]
    normsT = jnp.sqrt(XT * XT + YT * YT + 1e-12)
    r = jnp.maximum(base[None, :, :]
                    + XT[:, :, None] * wdx[None, None, :]
                    + YT[:, :, None] * wdy[None, None, :]
                    + normsT[:, :, None] * csum[None, None, :], 0.0)
    nf = jnp.concatenate([nyT, r], axis=-1)
    tn = nf.shape[1]
    z = nf.reshape(NEIGH * tn, 2 * D) @ Ws + bs[None, :]
    z = z - jnp.max(z, axis=1, keepdims=True)
    e = jnp.exp(z)
    scores = (e / jnp.sum(e, axis=1, keepdims=True)).reshape(NEIGH, tn, 2 * D)
    att = jnp.sum(nf * scores, axis=0)
    mlin, pear = _ggf(Xn, Yn)
    cat = jnp.concatenate([att, mlin, pear], axis=1)
    return jnp.maximum(cat @ Wf + bf[None, :], 0.0)


def _att0_kernel(xn_ref, yn_ref, xt_ref, yt_ref, ny_ref,
                 ax_ref, ay_ref, cs_ref, wdx_ref, wdy_ref, bl_ref,
                 ws_ref, bs_ref, wf_ref, bf_ref, o_ref):
    o_ref[0] = _block_body(xn_ref[0], yn_ref[0], xt_ref[0], yt_ref[0],
                           ny_ref[0], ax_ref[...], ay_ref[...], cs_ref[...],
                           wdx_ref[...], wdy_ref[...], bl_ref[...],
                           ws_ref[...], bs_ref[...], wf_ref[...], bf_ref[...])


def _att1_kernel(xn_ref, yn_ref, xt_ref, yt_ref, ny_ref,
                 ax_ref, ay_ref, cs_ref, wdx_ref, wdy_ref, bl_ref,
                 ws_ref, bs_ref, wf_ref, bf_ref,
                 w1_ref, b1_ref, skip_ref, o_ref):
    y2 = _block_body(xn_ref[0], yn_ref[0], xt_ref[0], yt_ref[0],
                     ny_ref[0], ax_ref[...], ay_ref[...], cs_ref[...],
                     wdx_ref[...], wdy_ref[...], bl_ref[...],
                     ws_ref[...], bs_ref[...], wf_ref[...], bf_ref[...])
    y3 = jnp.maximum(y2 @ w1_ref[...] + b1_ref[...], 0.0)
    z = y3 + skip_ref[0]
    o_ref[0] = jnp.where(z >= 0, z, LRELU_ALPHA * z)


def _wspecs(D):
    return [
        pl.BlockSpec((NEIGH, D), lambda b_, t: (0, 0)),
        pl.BlockSpec((NEIGH, D), lambda b_, t: (0, 0)),
        pl.BlockSpec((D,), lambda b_, t: (0,)),
        pl.BlockSpec((D,), lambda b_, t: (0,)),
        pl.BlockSpec((D,), lambda b_, t: (0,)),
        pl.BlockSpec((D,), lambda b_, t: (0,)),
        pl.BlockSpec((2 * D, 2 * D), lambda b_, t: (0, 0)),
        pl.BlockSpec((2 * D,), lambda b_, t: (0,)),
        pl.BlockSpec((2 * D + 2, 2 * D), lambda b_, t: (0, 0)),
        pl.BlockSpec((2 * D,), lambda b_, t: (0,)),
    ]


def _pt_specs(TN):
    return [
        pl.BlockSpec((1, TN, NEIGH), lambda b_, t: (b_, t, 0)),
        pl.BlockSpec((1, TN, NEIGH), lambda b_, t: (b_, t, 0)),
        pl.BlockSpec((1, NEIGH, TN), lambda b_, t: (b_, 0, t)),
        pl.BlockSpec((1, NEIGH, TN), lambda b_, t: (b_, 0, t)),
    ]


def _att_layer0(Xn, Yn, XT, YT, nyT, Wl0, bl0, Ws0, bs0, Wf0, bf0):
    D = UNITS // 4
    Ax, Ay, csum, wdx, wdy = _split_locse_w(Wl0)
    TN = TN_L
    return pl.pallas_call(
        _att0_kernel,
        grid=(B, N // TN),
        in_specs=_pt_specs(TN) + [
            pl.BlockSpec((1, NEIGH, TN, D), lambda b_, t: (b_, 0, t, 0)),
        ] + _wspecs(D),
        out_specs=pl.BlockSpec((1, TN, 2 * D), lambda b_, t: (b_, t, 0)),
        out_shape=jax.ShapeDtypeStruct((B, N, 2 * D), jnp.float32),
    )(Xn, Yn, XT, YT, nyT, Ax, Ay, csum, wdx, wdy, bl0, Ws0, bs0, Wf0, bf0)


def _att_layer1(Xn, Yn, XT, YT, nyT, Wl1, bl1, Ws1, bs1, Wf1, bf1, W1, b1, skip):
    D = UNITS // 2
    Ax, Ay, csum, wdx, wdy = _split_locse_w(Wl1)
    TN = TN_L
    return pl.pallas_call(
        _att1_kernel,
        grid=(B, N // TN),
        in_specs=_pt_specs(TN) + [
            pl.BlockSpec((1, NEIGH, TN, D), lambda b_, t: (b_, 0, t, 0)),
        ] + _wspecs(D) + [
            pl.BlockSpec((UNITS, UNITS), lambda b_, t: (0, 0)),
            pl.BlockSpec((UNITS,), lambda b_, t: (0,)),
            pl.BlockSpec((1, TN, UNITS), lambda b_, t: (b_, t, 0)),
        ],
        out_specs=pl.BlockSpec((1, TN, UNITS), lambda b_, t: (b_, t, 0)),
        out_shape=jax.ShapeDtypeStruct((B, N, UNITS), jnp.float32),
    )(Xn, Yn, XT, YT, nyT, Ax, Ay, csum, wdx, wdy, bl1, Ws1, bs1, Wf1, bf1,
      W1, b1, skip)


def _final_pallas(y, skip):
    TN = 512
    return pl.pallas_call(
        _final2_kernel,
        grid=(B, N // TN),
        in_specs=[
            pl.BlockSpec((1, TN, UNITS), lambda b_, n_: (b_, n_, 0)),
            pl.BlockSpec((1, TN, UNITS), lambda b_, n_: (b_, n_, 0)),
        ],
        out_specs=pl.BlockSpec((1, TN, UNITS), lambda b_, n_: (b_, n_, 0)),
        out_shape=jax.ShapeDtypeStruct((B, N, UNITS), jnp.float32),
    )(y, skip)


def kernel(pc, feats, W0, b0, Wl0, bl0, Ws0, bs0, Wf0, bf0, Wl1, bl1, Ws1, bs1, Wf1, bf1, W1, b1, Wskip, bskip):
    n_idx, y0, skip = _knn_mlp0(pc, feats, W0, b0, Wskip, bskip)
    npts = _gather_neighbours(pc, n_idx)
    Xn = npts[..., 0]
    Yn = npts[..., 1]
    XT = Xn.transpose(0, 2, 1)
    YT = Yn.transpose(0, 2, 1)
    nyT0 = _gather_neighbours(y0, n_idx).transpose(0, 2, 1, 3)
    y1 = _att_layer0(Xn, Yn, XT, YT, nyT0, Wl0, bl0, Ws0, bs0, Wf0, bf0)
    nyT1 = _gather_neighbours(y1, n_idx).transpose(0, 2, 1, 3)
    return _att_layer1(Xn, Yn, XT, YT, nyT1, Wl1, bl1, Ws1, bs1, Wf1, bf1,
                       W1, b1, skip)


# speedup vs baseline: 1.9421x; 1.0471x over previous
import jax
import jax.numpy as jnp
from jax.experimental import pallas as pl

B, N, DIMS = 4, 4096, 2
INPUT_UNITS = 64
UNITS = 128
K = 16
NEIGH = K + 1
LRELU_ALPHA = 0.2


TN_KNN = 512


def _knn_mlp0_kernel(pxc_ref, pyc_ref, pxr_ref, pyr_ref, feats_ref,
                     w0_ref, b0_ref, wsk_ref, bsk_ref,
                     idx_ref, y0_ref, skip_ref):
    xc = pxc_ref[0]
    yc = pyc_ref[0]
    xr = pxr_ref[0]
    yr = pyr_ref[0]
    dx = xc - xr
    dy = yc - yr
    d2 = dx * dx + dy * dy
    iota = jax.lax.broadcasted_iota(jnp.int32, (TN_KNN, N), 1)
    d2i = jax.lax.bitcast_convert_type(d2, jnp.int32)
    key = (d2i & jnp.int32(~0xFFF)) | iota
    cols = []
    for _ in range(NEIGH):
        m = jnp.min(key, axis=1, keepdims=True)
        key = jnp.where(key == m, jnp.int32(0x7FFFFFFF), key)
        cols.append(m & jnp.int32(0xFFF))
    idx_ref[0] = jnp.concatenate(cols, axis=1)
    f = feats_ref[0]
    y0_ref[0] = jnp.maximum(f @ w0_ref[...] + b0_ref[...], 0.0)
    skip_ref[0] = jnp.maximum(f @ wsk_ref[...] + bsk_ref[...], 0.0)


def _knn_mlp0(pc, feats, W0, b0, Wskip, bskip):
    pxc = pc[:, :, 0:1]
    pyc = pc[:, :, 1:2]
    pxr = pc[:, :, 0].reshape(B, 1, N)
    pyr = pc[:, :, 1].reshape(B, 1, N)
    G = N // TN_KNN
    return pl.pallas_call(
        _knn_mlp0_kernel,
        grid=(B, G),
        in_specs=[
            pl.BlockSpec((1, TN_KNN, 1), lambda b_, t: (b_, t, 0)),
            pl.BlockSpec((1, TN_KNN, 1), lambda b_, t: (b_, t, 0)),
            pl.BlockSpec((1, 1, N), lambda b_, t: (b_, 0, 0)),
            pl.BlockSpec((1, 1, N), lambda b_, t: (b_, 0, 0)),
            pl.BlockSpec((1, TN_KNN, INPUT_UNITS), lambda b_, t: (b_, t, 0)),
            pl.BlockSpec((INPUT_UNITS, UNITS // 4), lambda b_, t: (0, 0)),
            pl.BlockSpec((UNITS // 4,), lambda b_, t: (0,)),
            pl.BlockSpec((INPUT_UNITS, UNITS), lambda b_, t: (0, 0)),
            pl.BlockSpec((UNITS,), lambda b_, t: (0,)),
        ],
        out_specs=[
            pl.BlockSpec((1, TN_KNN, NEIGH), lambda b_, t: (b_, t, 0)),
            pl.BlockSpec((1, TN_KNN, UNITS // 4), lambda b_, t: (b_, t, 0)),
            pl.BlockSpec((1, TN_KNN, UNITS), lambda b_, t: (b_, t, 0)),
        ],
        out_shape=[
            jax.ShapeDtypeStruct((B, N, NEIGH), jnp.int32),
            jax.ShapeDtypeStruct((B, N, UNITS // 4), jnp.float32),
            jax.ShapeDtypeStruct((B, N, UNITS), jnp.float32),
        ],
    )(pxc, pyc, pxr, pyr, feats, W0, b0, Wskip, bskip)


def _gather_neighbours(x, n_idx):
    b, n, k = n_idx.shape
    def g(xx, ii):
        return xx[ii.reshape(-1)].reshape(n, k, xx.shape[-1])
    return jax.vmap(g)(x, n_idx)


def _m_lin_fit(pts):
    x = pts[..., 0]
    y = pts[..., 1]
    k = pts.shape[-2]
    sx = jnp.sum(x, -1)
    sy = jnp.sum(y, -1)
    num = k * jnp.sum(x * y, -1) - sx * sy
    den = k * jnp.sum(x * x, -1) - sx ** 2
    return (num / (den + 1e-8))[..., None, None]


def _pearson(pts):
    x = pts[..., 0]
    y = pts[..., 1]
    xm = x - x.mean(-1, keepdims=True)
    ym = y - y.mean(-1, keepdims=True)
    num = jnp.sum(xm * ym, -1)
    den = jnp.sqrt(jnp.sum(xm * xm, -1) * jnp.sum(ym * ym, -1)) + 1e-8
    return (num / den)[..., None, None]


def _locse(pc, feats, n_idx, W, b):
    n_points = _gather_neighbours(pc, n_idx)
    norms = jnp.sqrt(jnp.sum(n_points ** 2, axis=-1, keepdims=True) + 1e-12)
    rpbns = []
    for i in range(NEIGH):
        current = n_points[:, :, i:i + 1]
        diff = current - n_points
        rpbns.append(jnp.concatenate([diff, norms], axis=-1))
    rppe = jnp.concatenate([n_points] + rpbns, axis=-1)
    ggf = jnp.concatenate([_m_lin_fit(n_points), _pearson(n_points)], axis=-1)
    r = jax.nn.relu(jnp.einsum('bnkc,cd->bnkd', rppe, W) + b)
    n_feats = _gather_neighbours(feats, n_idx)
    return jnp.concatenate([n_feats, r], axis=-1), ggf


def _att_pool(n_feats, ggf, Ws, bs, Wf, bf):
    scores = jax.nn.softmax(jnp.einsum('bnkc,cd->bnkd', n_feats, Ws) + bs, axis=-1)
    attention = jnp.sum(n_feats * scores, axis=-2, keepdims=True)
    cat = jnp.concatenate([attention, ggf], axis=-1)
    out = jax.nn.relu(jnp.einsum('bnkc,cd->bnkd', cat, Wf) + bf)
    return out.reshape(out.shape[0], out.shape[1], -1)


TN_L = 512


def _split_locse_w(Wl):
    Ax = Wl[2::3]
    Ay = Wl[3::3]
    Cn = Wl[4::3]
    csum = jnp.sum(Cn, axis=0)
    wdx = Wl[0] - jnp.sum(Ax, axis=0)
    wdy = Wl[1] - jnp.sum(Ay, axis=0)
    return Ax, Ay, csum, wdx, wdy


def _ggf(Xn, Yn):
    k = float(NEIGH)
    sx = jnp.sum(Xn, 1, keepdims=True)
    sy = jnp.sum(Yn, 1, keepdims=True)
    sxy = jnp.sum(Xn * Yn, 1, keepdims=True)
    sxx = jnp.sum(Xn * Xn, 1, keepdims=True)
    mlin = (k * sxy - sx * sy) / (k * sxx - sx * sx + 1e-8)
    xm = Xn - sx / k
    ym = Yn - sy / k
    num = jnp.sum(xm * ym, 1, keepdims=True)
    den = jnp.sqrt(jnp.sum(xm * xm, 1, keepdims=True) *
                   jnp.sum(ym * ym, 1, keepdims=True)) + 1e-8
    return mlin, num / den


def _block_body(Xn, Yn, XT, YT, nyT, Ax, Ay, csum, wdx, wdy, bl, Ws, bs, Wf, bf):
    D = Ax.shape[1]
    base = Xn @ Ax + Yn @ Ay + bl[None, :]
    normsT = jnp.sqrt(XT * XT + YT * YT + 1e-12)
    r = jnp.maximum(base[None, :, :]
                    + XT[:, :, None] * wdx[None, None, :]
                    + YT[:, :, None] * wdy[None, None, :]
                    + normsT[:, :, None] * csum[None, None, :], 0.0)
    nf = jnp.concatenate([nyT, r], axis=-1)
    tn = nf.shape[1]
    z = nf.reshape(NEIGH * tn, 2 * D) @ Ws + bs[None, :]
    z = z - jnp.max(z, axis=1, keepdims=True)
    e = jnp.exp(z)
    scores = (e / jnp.sum(e, axis=1, keepdims=True)).reshape(NEIGH, tn, 2 * D)
    att = jnp.sum(nf * scores, axis=0)
    mlin, pear = _ggf(Xn, Yn)
    cat = jnp.concatenate([att, mlin, pear], axis=1)
    return jnp.maximum(cat @ Wf + bf[None, :], 0.0)


def _att0_kernel(xn_ref, yn_ref, xt_ref, yt_ref, ny_ref,
                 ax_ref, ay_ref, cs_ref, wdx_ref, wdy_ref, bl_ref,
                 ws_ref, bs_ref, wf_ref, bf_ref, o_ref):
    o_ref[0] = _block_body(xn_ref[0], yn_ref[0], xt_ref[0], yt_ref[0],
                           ny_ref[0], ax_ref[...], ay_ref[...], cs_ref[...],
                           wdx_ref[...], wdy_ref[...], bl_ref[...],
                           ws_ref[...], bs_ref[...], wf_ref[...], bf_ref[...])


def _att1_kernel(xn_ref, yn_ref, xt_ref, yt_ref, ny_ref,
                 ax_ref, ay_ref, cs_ref, wdx_ref, wdy_ref, bl_ref,
                 ws_ref, bs_ref, wf_ref, bf_ref,
                 w1_ref, b1_ref, skip_ref, o_ref):
    y2 = _block_body(xn_ref[0], yn_ref[0], xt_ref[0], yt_ref[0],
                     ny_ref[0], ax_ref[...], ay_ref[...], cs_ref[...],
                     wdx_ref[...], wdy_ref[...], bl_ref[...],
                     ws_ref[...], bs_ref[...], wf_ref[...], bf_ref[...])
    y3 = jnp.maximum(y2 @ w1_ref[...] + b1_ref[...], 0.0)
    z = y3 + skip_ref[0]
    o_ref[0] = jnp.where(z >= 0, z, LRELU_ALPHA * z)


def _wspecs(D):
    return [
        pl.BlockSpec((NEIGH, D), lambda b_, t: (0, 0)),
        pl.BlockSpec((NEIGH, D), lambda b_, t: (0, 0)),
        pl.BlockSpec((D,), lambda b_, t: (0,)),
        pl.BlockSpec((D,), lambda b_, t: (0,)),
        pl.BlockSpec((D,), lambda b_, t: (0,)),
        pl.BlockSpec((D,), lambda b_, t: (0,)),
        pl.BlockSpec((2 * D, 2 * D), lambda b_, t: (0, 0)),
        pl.BlockSpec((2 * D,), lambda b_, t: (0,)),
        pl.BlockSpec((2 * D + 2, 2 * D), lambda b_, t: (0, 0)),
        pl.BlockSpec((2 * D,), lambda b_, t: (0,)),
    ]


def _pt_specs(TN):
    return [
        pl.BlockSpec((1, TN, NEIGH), lambda b_, t: (b_, t, 0)),
        pl.BlockSpec((1, TN, NEIGH), lambda b_, t: (b_, t, 0)),
        pl.BlockSpec((1, NEIGH, TN), lambda b_, t: (b_, 0, t)),
        pl.BlockSpec((1, NEIGH, TN), lambda b_, t: (b_, 0, t)),
    ]


def _att_layer0(Xn, Yn, XT, YT, nyT, Wl0, bl0, Ws0, bs0, Wf0, bf0):
    D = UNITS // 4
    Ax, Ay, csum, wdx, wdy = _split_locse_w(Wl0)
    TN = TN_L
    return pl.pallas_call(
        _att0_kernel,
        grid=(B, N // TN),
        in_specs=_pt_specs(TN) + [
            pl.BlockSpec((1, NEIGH, TN, D), lambda b_, t: (b_, 0, t, 0)),
        ] + _wspecs(D),
        out_specs=pl.BlockSpec((1, TN, 2 * D), lambda b_, t: (b_, t, 0)),
        out_shape=jax.ShapeDtypeStruct((B, N, 2 * D), jnp.float32),
    )(Xn, Yn, XT, YT, nyT, Ax, Ay, csum, wdx, wdy, bl0, Ws0, bs0, Wf0, bf0)


def _att_layer1(Xn, Yn, XT, YT, nyT, Wl1, bl1, Ws1, bs1, Wf1, bf1, W1, b1, skip):
    D = UNITS // 2
    Ax, Ay, csum, wdx, wdy = _split_locse_w(Wl1)
    TN = TN_L
    return pl.pallas_call(
        _att1_kernel,
        grid=(B, N // TN),
        in_specs=_pt_specs(TN) + [
            pl.BlockSpec((1, NEIGH, TN, D), lambda b_, t: (b_, 0, t, 0)),
        ] + _wspecs(D) + [
            pl.BlockSpec((UNITS, UNITS), lambda b_, t: (0, 0)),
            pl.BlockSpec((UNITS,), lambda b_, t: (0,)),
            pl.BlockSpec((1, TN, UNITS), lambda b_, t: (b_, t, 0)),
        ],
        out_specs=pl.BlockSpec((1, TN, UNITS), lambda b_, t: (b_, t, 0)),
        out_shape=jax.ShapeDtypeStruct((B, N, UNITS), jnp.float32),
    )(Xn, Yn, XT, YT, nyT, Ax, Ay, csum, wdx, wdy, bl1, Ws1, bs1, Wf1, bf1,
      W1, b1, skip)


def _final_pallas(y, skip):
    TN = 512
    return pl.pallas_call(
        _final2_kernel,
        grid=(B, N // TN),
        in_specs=[
            pl.BlockSpec((1, TN, UNITS), lambda b_, n_: (b_, n_, 0)),
            pl.BlockSpec((1, TN, UNITS), lambda b_, n_: (b_, n_, 0)),
        ],
        out_specs=pl.BlockSpec((1, TN, UNITS), lambda b_, n_: (b_, n_, 0)),
        out_shape=jax.ShapeDtypeStruct((B, N, UNITS), jnp.float32),
    )(y, skip)


def kernel(pc, feats, W0, b0, Wl0, bl0, Ws0, bs0, Wf0, bf0, Wl1, bl1, Ws1, bs1, Wf1, bf1, W1, b1, Wskip, bskip):
    n_idx, y0, skip = _knn_mlp0(pc, feats, W0, b0, Wskip, bskip)
    npts = _gather_neighbours(pc, n_idx)
    Xn = npts[..., 0]
    Yn = npts[..., 1]
    XT = Xn.transpose(0, 2, 1)
    YT = Yn.transpose(0, 2, 1)
    nyT0 = _gather_neighbours(y0, n_idx).transpose(0, 2, 1, 3)
    y1 = _att_layer0(Xn, Yn, XT, YT, nyT0, Wl0, bl0, Ws0, bs0, Wf0, bf0)
    nyT1 = _gather_neighbours(y1, n_idx).transpose(0, 2, 1, 3)
    return _att_layer1(Xn, Yn, XT, YT, nyT1, Wl1, bl1, Ws1, bs1, Wf1, bf1,
                       W1, b1, skip)


# speedup vs baseline: 16.2927x; 8.3892x over previous
import functools

import jax
import jax.numpy as jnp
from jax import lax
from jax.experimental import pallas as pl
from jax.experimental.pallas import tpu as pltpu
from jax.experimental.pallas import tpu_sc as plsc

B, N, DIMS = 4, 4096, 2
INPUT_UNITS = 64
UNITS = 128
K = 16
NEIGH = K + 1
LRELU_ALPHA = 0.2


TN_KNN = 512


def _knn_mlp0_kernel(pxc_ref, pyc_ref, pxr_ref, pyr_ref, feats_ref,
                     w0_ref, b0_ref, wsk_ref, bsk_ref,
                     idx_ref, y0_ref, skip_ref):
    xc = pxc_ref[0]
    yc = pyc_ref[0]
    xr = pxr_ref[0]
    yr = pyr_ref[0]
    dx = xc - xr
    dy = yc - yr
    d2 = dx * dx + dy * dy
    iota = jax.lax.broadcasted_iota(jnp.int32, (TN_KNN, N), 1)
    d2i = jax.lax.bitcast_convert_type(d2, jnp.int32)
    key = (d2i & jnp.int32(~0xFFF)) | iota
    cols = []
    for _ in range(NEIGH):
        m = jnp.min(key, axis=1, keepdims=True)
        key = jnp.where(key == m, jnp.int32(0x7FFFFFFF), key)
        cols.append(m & jnp.int32(0xFFF))
    idx_ref[0] = jnp.concatenate(cols, axis=1)
    f = feats_ref[0]
    y0_ref[0] = jnp.maximum(f @ w0_ref[...] + b0_ref[...], 0.0)
    skip_ref[0] = jnp.maximum(f @ wsk_ref[...] + bsk_ref[...], 0.0)


def _knn_mlp0(pc, feats, W0, b0, Wskip, bskip):
    pxc = pc[:, :, 0:1]
    pyc = pc[:, :, 1:2]
    pxr = pc[:, :, 0].reshape(B, 1, N)
    pyr = pc[:, :, 1].reshape(B, 1, N)
    G = N // TN_KNN
    return pl.pallas_call(
        _knn_mlp0_kernel,
        grid=(B, G),
        in_specs=[
            pl.BlockSpec((1, TN_KNN, 1), lambda b_, t: (b_, t, 0)),
            pl.BlockSpec((1, TN_KNN, 1), lambda b_, t: (b_, t, 0)),
            pl.BlockSpec((1, 1, N), lambda b_, t: (b_, 0, 0)),
            pl.BlockSpec((1, 1, N), lambda b_, t: (b_, 0, 0)),
            pl.BlockSpec((1, TN_KNN, INPUT_UNITS), lambda b_, t: (b_, t, 0)),
            pl.BlockSpec((INPUT_UNITS, UNITS // 4), lambda b_, t: (0, 0)),
            pl.BlockSpec((UNITS // 4,), lambda b_, t: (0,)),
            pl.BlockSpec((INPUT_UNITS, UNITS), lambda b_, t: (0, 0)),
            pl.BlockSpec((UNITS,), lambda b_, t: (0,)),
        ],
        out_specs=[
            pl.BlockSpec((1, TN_KNN, NEIGH), lambda b_, t: (b_, t, 0)),
            pl.BlockSpec((1, TN_KNN, UNITS // 4), lambda b_, t: (b_, t, 0)),
            pl.BlockSpec((1, TN_KNN, UNITS), lambda b_, t: (b_, t, 0)),
        ],
        out_shape=[
            jax.ShapeDtypeStruct((B, N, NEIGH), jnp.int32),
            jax.ShapeDtypeStruct((B, N, UNITS // 4), jnp.float32),
            jax.ShapeDtypeStruct((B, N, UNITS), jnp.float32),
        ],
    )(pxc, pyc, pxr, pyr, feats, W0, b0, Wskip, bskip)


def _gather_neighbours(x, n_idx):
    b, n, k = n_idx.shape
    def g(xx, ii):
        return xx[ii.reshape(-1)].reshape(n, k, xx.shape[-1])
    return jax.vmap(g)(x, n_idx)


def _m_lin_fit(pts):
    x = pts[..., 0]
    y = pts[..., 1]
    k = pts.shape[-2]
    sx = jnp.sum(x, -1)
    sy = jnp.sum(y, -1)
    num = k * jnp.sum(x * y, -1) - sx * sy
    den = k * jnp.sum(x * x, -1) - sx ** 2
    return (num / (den + 1e-8))[..., None, None]


def _pearson(pts):
    x = pts[..., 0]
    y = pts[..., 1]
    xm = x - x.mean(-1, keepdims=True)
    ym = y - y.mean(-1, keepdims=True)
    num = jnp.sum(xm * ym, -1)
    den = jnp.sqrt(jnp.sum(xm * xm, -1) * jnp.sum(ym * ym, -1)) + 1e-8
    return (num / den)[..., None, None]


def _locse(pc, feats, n_idx, W, b):
    n_points = _gather_neighbours(pc, n_idx)
    norms = jnp.sqrt(jnp.sum(n_points ** 2, axis=-1, keepdims=True) + 1e-12)
    rpbns = []
    for i in range(NEIGH):
        current = n_points[:, :, i:i + 1]
        diff = current - n_points
        rpbns.append(jnp.concatenate([diff, norms], axis=-1))
    rppe = jnp.concatenate([n_points] + rpbns, axis=-1)
    ggf = jnp.concatenate([_m_lin_fit(n_points), _pearson(n_points)], axis=-1)
    r = jax.nn.relu(jnp.einsum('bnkc,cd->bnkd', rppe, W) + b)
    n_feats = _gather_neighbours(feats, n_idx)
    return jnp.concatenate([n_feats, r], axis=-1), ggf


def _att_pool(n_feats, ggf, Ws, bs, Wf, bf):
    scores = jax.nn.softmax(jnp.einsum('bnkc,cd->bnkd', n_feats, Ws) + bs, axis=-1)
    attention = jnp.sum(n_feats * scores, axis=-2, keepdims=True)
    cat = jnp.concatenate([attention, ggf], axis=-1)
    out = jax.nn.relu(jnp.einsum('bnkc,cd->bnkd', cat, Wf) + bf)
    return out.reshape(out.shape[0], out.shape[1], -1)


M_IDX = B * NEIGH * N
NW_SC = 32
MPW = M_IDX // NW_SC
CH_SC = 128


def _sc_gather_pc_y0(idx, pcpad, t0):
    D0 = t0.shape[1]
    DP = pcpad.shape[1]
    mesh = plsc.VectorSubcoreMesh(core_axis_name="c", subcore_axis_name="s")

    @functools.partial(
        pl.kernel, mesh=mesh,
        compiler_params=pltpu.CompilerParams(use_tc_tiling_on_sc=False),
        out_type=[
            jax.ShapeDtypeStruct((M_IDX, DP), jnp.float32),
            jax.ShapeDtypeStruct((M_IDX, D0), jnp.float32),
        ],
        scratch_types=[
            pltpu.VMEM((MPW,), jnp.int32),
            pltpu.VMEM((CH_SC, DP), jnp.float32),
            pltpu.VMEM((CH_SC, D0), jnp.float32),
            pltpu.SemaphoreType.DMA,
        ],
    )
    def kb(idx_hbm, pc_hbm, t0_hbm, pcg_hbm, ny0_hbm,
           idx_v, prow_v, rows_v, sem):
        wid = lax.axis_index("s") * 2 + lax.axis_index("c")
        base = wid * MPW
        pltpu.sync_copy(idx_hbm.at[pl.ds(base, MPW)], idx_v)

        def body2(c, carry):
            cp1 = pltpu.async_copy(
                pc_hbm.at[idx_v.at[pl.ds(c * CH_SC, CH_SC)]], prow_v, sem)
            cp2 = pltpu.async_copy(
                t0_hbm.at[idx_v.at[pl.ds(c * CH_SC, CH_SC)]], rows_v, sem)
            cp1.wait()
            cp2.wait()
            pltpu.sync_copy(prow_v, pcg_hbm.at[pl.ds(base + c * CH_SC, CH_SC)])
            pltpu.sync_copy(rows_v, ny0_hbm.at[pl.ds(base + c * CH_SC, CH_SC)])
            return carry

        lax.fori_loop(0, MPW // CH_SC, body2, 0)

    return kb(idx, pcpad, t0)


def _sc_gather_rows(idx, t1):
    D1 = t1.shape[1]
    mesh = plsc.VectorSubcoreMesh(core_axis_name="c", subcore_axis_name="s")

    @functools.partial(
        pl.kernel, mesh=mesh,
        compiler_params=pltpu.CompilerParams(use_tc_tiling_on_sc=False),
        out_type=jax.ShapeDtypeStruct((M_IDX, D1), jnp.float32),
        scratch_types=[
            pltpu.VMEM((MPW,), jnp.int32),
            pltpu.VMEM((CH_SC, D1), jnp.float32),
            pltpu.SemaphoreType.DMA,
        ],
    )
    def kd(idx_hbm, t1_hbm, ny1_hbm, idx_v, rows_v, sem):
        wid = lax.axis_index("s") * 2 + lax.axis_index("c")
        base = wid * MPW
        pltpu.sync_copy(idx_hbm.at[pl.ds(base, MPW)], idx_v)

        def body2(c, carry):
            pltpu.async_copy(t1_hbm.at[idx_v.at[pl.ds(c * CH_SC, CH_SC)]],
                             rows_v, sem).wait()
            pltpu.sync_copy(rows_v, ny1_hbm.at[pl.ds(base + c * CH_SC, CH_SC)])
            return carry

        lax.fori_loop(0, MPW // CH_SC, body2, 0)

    return kd(idx, t1)


TN_L = 512


def _split_locse_w(Wl):
    Ax = Wl[2::3]
    Ay = Wl[3::3]
    Cn = Wl[4::3]
    csum = jnp.sum(Cn, axis=0)
    wdx = Wl[0] - jnp.sum(Ax, axis=0)
    wdy = Wl[1] - jnp.sum(Ay, axis=0)
    return Ax, Ay, csum, wdx, wdy


def _ggf(Xn, Yn):
    k = float(NEIGH)
    sx = jnp.sum(Xn, 1, keepdims=True)
    sy = jnp.sum(Yn, 1, keepdims=True)
    sxy = jnp.sum(Xn * Yn, 1, keepdims=True)
    sxx = jnp.sum(Xn * Xn, 1, keepdims=True)
    mlin = (k * sxy - sx * sy) / (k * sxx - sx * sx + 1e-8)
    xm = Xn - sx / k
    ym = Yn - sy / k
    num = jnp.sum(xm * ym, 1, keepdims=True)
    den = jnp.sqrt(jnp.sum(xm * xm, 1, keepdims=True) *
                   jnp.sum(ym * ym, 1, keepdims=True)) + 1e-8
    return mlin, num / den


def _block_body(Xn, Yn, XT, YT, nyT, Ax, Ay, csum, wdx, wdy, bl, Ws, bs, Wf, bf):
    D = Ax.shape[1]
    base = Xn @ Ax + Yn @ Ay + bl[None, :]
    normsT = jnp.sqrt(XT * XT + YT * YT + 1e-12)
    r = jnp.maximum(base[None, :, :]
                    + XT[:, :, None] * wdx[None, None, :]
                    + YT[:, :, None] * wdy[None, None, :]
                    + normsT[:, :, None] * csum[None, None, :], 0.0)
    nf = jnp.concatenate([nyT, r], axis=-1)
    tn = nf.shape[1]
    z = nf.reshape(NEIGH * tn, 2 * D) @ Ws + bs[None, :]
    z = z - jnp.max(z, axis=1, keepdims=True)
    e = jnp.exp(z)
    scores = (e / jnp.sum(e, axis=1, keepdims=True)).reshape(NEIGH, tn, 2 * D)
    att = jnp.sum(nf * scores, axis=0)
    mlin, pear = _ggf(Xn, Yn)
    cat = jnp.concatenate([att, mlin, pear], axis=1)
    return jnp.maximum(cat @ Wf + bf[None, :], 0.0)


def _att0_kernel(xn_ref, yn_ref, xt_ref, yt_ref, ny_ref,
                 ax_ref, ay_ref, cs_ref, wdx_ref, wdy_ref, bl_ref,
                 ws_ref, bs_ref, wf_ref, bf_ref, o_ref):
    o_ref[0] = _block_body(xn_ref[0], yn_ref[0], xt_ref[0], yt_ref[0],
                           ny_ref[0], ax_ref[...], ay_ref[...], cs_ref[...],
                           wdx_ref[...], wdy_ref[...], bl_ref[...],
                           ws_ref[...], bs_ref[...], wf_ref[...], bf_ref[...])


def _att1_kernel(xn_ref, yn_ref, xt_ref, yt_ref, ny_ref,
                 ax_ref, ay_ref, cs_ref, wdx_ref, wdy_ref, bl_ref,
                 ws_ref, bs_ref, wf_ref, bf_ref,
                 w1_ref, b1_ref, skip_ref, o_ref):
    y2 = _block_body(xn_ref[0], yn_ref[0], xt_ref[0], yt_ref[0],
                     ny_ref[0], ax_ref[...], ay_ref[...], cs_ref[...],
                     wdx_ref[...], wdy_ref[...], bl_ref[...],
                     ws_ref[...], bs_ref[...], wf_ref[...], bf_ref[...])
    y3 = jnp.maximum(y2 @ w1_ref[...] + b1_ref[...], 0.0)
    z = y3 + skip_ref[0]
    o_ref[0] = jnp.where(z >= 0, z, LRELU_ALPHA * z)


def _wspecs(D):
    return [
        pl.BlockSpec((NEIGH, D), lambda b_, t: (0, 0)),
        pl.BlockSpec((NEIGH, D), lambda b_, t: (0, 0)),
        pl.BlockSpec((D,), lambda b_, t: (0,)),
        pl.BlockSpec((D,), lambda b_, t: (0,)),
        pl.BlockSpec((D,), lambda b_, t: (0,)),
        pl.BlockSpec((D,), lambda b_, t: (0,)),
        pl.BlockSpec((2 * D, 2 * D), lambda b_, t: (0, 0)),
        pl.BlockSpec((2 * D,), lambda b_, t: (0,)),
        pl.BlockSpec((2 * D + 2, 2 * D), lambda b_, t: (0, 0)),
        pl.BlockSpec((2 * D,), lambda b_, t: (0,)),
    ]


def _pt_specs(TN):
    return [
        pl.BlockSpec((1, TN, NEIGH), lambda b_, t: (b_, t, 0)),
        pl.BlockSpec((1, TN, NEIGH), lambda b_, t: (b_, t, 0)),
        pl.BlockSpec((1, NEIGH, TN), lambda b_, t: (b_, 0, t)),
        pl.BlockSpec((1, NEIGH, TN), lambda b_, t: (b_, 0, t)),
    ]


def _att_layer0(Xn, Yn, XT, YT, nyT, Wl0, bl0, Ws0, bs0, Wf0, bf0):
    D = UNITS // 4
    Ax, Ay, csum, wdx, wdy = _split_locse_w(Wl0)
    TN = TN_L
    return pl.pallas_call(
        _att0_kernel,
        grid=(B, N // TN),
        in_specs=_pt_specs(TN) + [
            pl.BlockSpec((1, NEIGH, TN, D), lambda b_, t: (b_, 0, t, 0)),
        ] + _wspecs(D),
        out_specs=pl.BlockSpec((1, TN, 2 * D), lambda b_, t: (b_, t, 0)),
        out_shape=jax.ShapeDtypeStruct((B, N, 2 * D), jnp.float32),
    )(Xn, Yn, XT, YT, nyT, Ax, Ay, csum, wdx, wdy, bl0, Ws0, bs0, Wf0, bf0)


def _att_layer1(Xn, Yn, XT, YT, nyT, Wl1, bl1, Ws1, bs1, Wf1, bf1, W1, b1, skip):
    D = UNITS // 2
    Ax, Ay, csum, wdx, wdy = _split_locse_w(Wl1)
    TN = TN_L
    return pl.pallas_call(
        _att1_kernel,
        grid=(B, N // TN),
        in_specs=_pt_specs(TN) + [
            pl.BlockSpec((1, NEIGH, TN, D), lambda b_, t: (b_, 0, t, 0)),
        ] + _wspecs(D) + [
            pl.BlockSpec((UNITS, UNITS), lambda b_, t: (0, 0)),
            pl.BlockSpec((UNITS,), lambda b_, t: (0,)),
            pl.BlockSpec((1, TN, UNITS), lambda b_, t: (b_, t, 0)),
        ],
        out_specs=pl.BlockSpec((1, TN, UNITS), lambda b_, t: (b_, t, 0)),
        out_shape=jax.ShapeDtypeStruct((B, N, UNITS), jnp.float32),
    )(Xn, Yn, XT, YT, nyT, Ax, Ay, csum, wdx, wdy, bl1, Ws1, bs1, Wf1, bf1,
      W1, b1, skip)


def _final_pallas(y, skip):
    TN = 512
    return pl.pallas_call(
        _final2_kernel,
        grid=(B, N // TN),
        in_specs=[
            pl.BlockSpec((1, TN, UNITS), lambda b_, n_: (b_, n_, 0)),
            pl.BlockSpec((1, TN, UNITS), lambda b_, n_: (b_, n_, 0)),
        ],
        out_specs=pl.BlockSpec((1, TN, UNITS), lambda b_, n_: (b_, n_, 0)),
        out_shape=jax.ShapeDtypeStruct((B, N, UNITS), jnp.float32),
    )(y, skip)


def kernel(pc, feats, W0, b0, Wl0, bl0, Ws0, bs0, Wf0, bf0, Wl1, bl1, Ws1, bs1, Wf1, bf1, W1, b1, Wskip, bskip):
    n_idx, y0, skip = _knn_mlp0(pc, feats, W0, b0, Wskip, bskip)
    idxg = n_idx + (jnp.arange(B, dtype=jnp.int32) * N)[:, None, None]
    idxT = idxg.transpose(0, 2, 1).reshape(-1)
    pcpad = jnp.pad(pc.reshape(B * N, DIMS), ((0, 0), (0, 14)))
    pcg, ny0 = _sc_gather_pc_y0(idxT, pcpad, y0.reshape(B * N, UNITS // 4))
    XT = pcg[:, 0].reshape(B, NEIGH, N)
    YT = pcg[:, 1].reshape(B, NEIGH, N)
    Xn = XT.transpose(0, 2, 1)
    Yn = YT.transpose(0, 2, 1)
    nyT0 = ny0.reshape(B, NEIGH, N, UNITS // 4)
    y1 = _att_layer0(Xn, Yn, XT, YT, nyT0, Wl0, bl0, Ws0, bs0, Wf0, bf0)
    ny1 = _sc_gather_rows(idxT, y1.reshape(B * N, UNITS // 2))
    nyT1 = ny1.reshape(B, NEIGH, N, UNITS // 2)
    return _att_layer1(Xn, Yn, XT, YT, nyT1, Wl1, bl1, Ws1, bs1, Wf1, bf1,
                       W1, b1, skip)
